# baseline (speedup 1.0000x reference)
import jax
import jax.numpy as jnp
from jax.experimental import pallas as pl
from jax.experimental.pallas import tpu as pltpu

_B, _C, _H, _W = 64, 3, 256, 256
_G = 16


def _f32_to_f16_bits_hi(y):
    u = jax.lax.bitcast_convert_type(y, jnp.int32)
    mag = u & jnp.int32(0x7FFFFFFF)
    rne = mag + jnp.int32(0xFFF) + ((mag >> 13) & jnp.int32(1))
    t16 = (rne >> 13) - jnp.int32(0x1C000)
    sgn = (u >> 16) & jnp.int32(0x8000)
    h = jnp.where(mag >= jnp.int32(0x38800000), t16, jnp.int32(0)) | sgn
    return jax.lax.bitcast_convert_type(h << 16, jnp.float32)


def _scalar_col(tab_ref, ts_ref, base):
    l = jax.lax.broadcasted_iota(jnp.int32, (_G, 1, 1, 1), 0)
    col = jnp.full((_G, 1, 1, 1), tab_ref[ts_ref[base]], dtype=jnp.float32)
    for i in range(1, _G):
        col = jnp.where(l == i, tab_ref[ts_ref[base + i]], col)
    return col


def _body(ts_ref, acp_ref, omacp_ref, lat_ref, noise_ref, out_ref):
    base = pl.program_id(0) * _G
    s1 = _scalar_col(acp_ref, ts_ref, base)
    s2 = _scalar_col(omacp_ref, ts_ref, base)
    y = lat_ref[...] * s1 + noise_ref[...] * s2
    zf = _f32_to_f16_bits_hi(y)
    out_ref.bitcast(jnp.bfloat16)[...] = zf.astype(jnp.bfloat16)


def kernel(latent, noise, timestep, sqrt_alphas_cum_prod, sqrt_one_minus_alphas_cum_prod):
    ts = timestep.astype(jnp.int32)
    acp = sqrt_alphas_cum_prod.astype(jnp.float16).astype(jnp.float32)
    omacp = sqrt_one_minus_alphas_cum_prod.astype(jnp.float16).astype(jnp.float32)

    grid_spec = pltpu.PrefetchScalarGridSpec(
        num_scalar_prefetch=3,
        grid=(_B // _G,),
        in_specs=[
            pl.BlockSpec((_G, _C, _H, _W), lambda b, *_: (b, 0, 0, 0)),
            pl.BlockSpec((_G, _C, _H, _W), lambda b, *_: (b, 0, 0, 0)),
        ],
        out_specs=pl.BlockSpec((_G, _C, _H, _W), lambda b, *_: (b, 0, 0, 0)),
    )
    out = pl.pallas_call(
        _body,
        grid_spec=grid_spec,
        out_shape=jax.ShapeDtypeStruct((_B, _C, _H, _W), jnp.float16),
        compiler_params=pltpu.CompilerParams(
            dimension_semantics=("parallel",),
            vmem_limit_bytes=100 * 1024 * 1024,
        ),
    )(ts, acp, omacp, latent, noise)
    return out


# speedup vs baseline: 1.0507x; 1.0507x over previous
import jax
import jax.numpy as jnp
from jax.experimental import pallas as pl
from jax.experimental.pallas import tpu as pltpu

_B, _C, _H, _W = 64, 3, 256, 256
_G = 8


def _f32_to_f16_bits_hi(y):
    u = jax.lax.bitcast_convert_type(y, jnp.int32)
    mag = u & jnp.int32(0x7FFFFFFF)
    rne = mag + jnp.int32(0x1000)
    t16 = (rne >> 13) - jnp.int32(0x1C000)
    sgn = (u >> 16) & jnp.int32(0x8000)
    h = jnp.where(mag >= jnp.int32(0x38800000), t16, jnp.int32(0)) | sgn
    return jax.lax.bitcast_convert_type(h << 16, jnp.float32)


def _scalar_col(tab_ref, ts_ref, base):
    l = jax.lax.broadcasted_iota(jnp.int32, (_G, 1, 1, 1), 0)
    col = jnp.full((_G, 1, 1, 1), tab_ref[ts_ref[base]], dtype=jnp.float32)
    for i in range(1, _G):
        col = jnp.where(l == i, tab_ref[ts_ref[base + i]], col)
    return col


def _body(ts_ref, acp_ref, omacp_ref, lat_ref, noise_ref, out_ref):
    base = pl.program_id(0) * _G
    s1 = _scalar_col(acp_ref, ts_ref, base)
    s2 = _scalar_col(omacp_ref, ts_ref, base)
    y = lat_ref[...] * s1 + noise_ref[...] * s2
    zf = _f32_to_f16_bits_hi(y)
    out_ref.bitcast(jnp.bfloat16)[...] = zf.astype(jnp.bfloat16)


def kernel(latent, noise, timestep, sqrt_alphas_cum_prod, sqrt_one_minus_alphas_cum_prod):
    ts = timestep.astype(jnp.int32)
    acp = sqrt_alphas_cum_prod.astype(jnp.float16).astype(jnp.float32)
    omacp = sqrt_one_minus_alphas_cum_prod.astype(jnp.float16).astype(jnp.float32)

    grid_spec = pltpu.PrefetchScalarGridSpec(
        num_scalar_prefetch=3,
        grid=(_B // _G,),
        in_specs=[
            pl.BlockSpec((_G, _C, _H, _W), lambda b, *_: (b, 0, 0, 0)),
            pl.BlockSpec((_G, _C, _H, _W), lambda b, *_: (b, 0, 0, 0)),
        ],
        out_specs=pl.BlockSpec((_G, _C, _H, _W), lambda b, *_: (b, 0, 0, 0)),
    )
    out = pl.pallas_call(
        _body,
        grid_spec=grid_spec,
        out_shape=jax.ShapeDtypeStruct((_B, _C, _H, _W), jnp.float16),
        compiler_params=pltpu.CompilerParams(
            dimension_semantics=("parallel",),
            vmem_limit_bytes=100 * 1024 * 1024,
        ),
    )(ts, acp, omacp, latent, noise)
    return out


# speedup vs baseline: 1.0514x; 1.0007x over previous
import jax
import jax.numpy as jnp
from jax.experimental import pallas as pl
from jax.experimental.pallas import tpu as pltpu

_B, _C, _H, _W = 64, 3, 256, 256
_G = 8


def _f32_to_f16_bits_hi(y):
    u = jax.lax.bitcast_convert_type(y, jnp.int32)
    mag = u & jnp.int32(0x7FFFFFFF)
    rne = mag + jnp.int32(0x1000)
    t16 = (rne >> 13) - jnp.int32(0x1C000)
    sgn = (u >> 16) & jnp.int32(0x8000)
    h = jnp.where(mag >= jnp.int32(0x38800000), t16, jnp.int32(0)) | sgn
    return jax.lax.bitcast_convert_type(h << 16, jnp.float32)


def _scalar_col(tab_ref, ts_ref, base):
    l = jax.lax.broadcasted_iota(jnp.int32, (_G, 1, 1, 1), 0)
    col = jnp.full((_G, 1, 1, 1), tab_ref[ts_ref[base]], dtype=jnp.float32)
    for i in range(1, _G):
        col = jnp.where(l == i, tab_ref[ts_ref[base + i]], col)
    return col


def _body(ts_ref, acp_ref, omacp_ref, lat_ref, noise_ref, out_ref):
    base = pl.program_id(0) * _G
    s1 = _scalar_col(acp_ref, ts_ref, base)
    s2 = _scalar_col(omacp_ref, ts_ref, base)
    y = lat_ref[...] * s1 + noise_ref[...] * s2
    zf = _f32_to_f16_bits_hi(y)
    out_ref.bitcast(jnp.bfloat16)[...] = zf.astype(jnp.bfloat16)


def kernel(latent, noise, timestep, sqrt_alphas_cum_prod, sqrt_one_minus_alphas_cum_prod):
    ts = timestep.astype(jnp.int32)
    acp = sqrt_alphas_cum_prod.astype(jnp.float16).astype(jnp.float32)
    omacp = sqrt_one_minus_alphas_cum_prod.astype(jnp.float16).astype(jnp.float32)

    grid_spec = pltpu.PrefetchScalarGridSpec(
        num_scalar_prefetch=3,
        grid=(_B // _G,),
        in_specs=[
            pl.BlockSpec((_G, _C, _H, _W), lambda b, *_: (b, 0, 0, 0)),
            pl.BlockSpec((_G, _C, _H, _W), lambda b, *_: (b, 0, 0, 0)),
        ],
        out_specs=pl.BlockSpec((_G, _C, _H, _W), lambda b, *_: (b, 0, 0, 0)),
    )
    out = pl.pallas_call(
        _body,
        grid_spec=grid_spec,
        out_shape=jax.ShapeDtypeStruct((_B, _C, _H, _W), jnp.float16),
        compiler_params=pltpu.CompilerParams(
            dimension_semantics=("arbitrary",),
            vmem_limit_bytes=100 * 1024 * 1024,
        ),
    )(ts, acp, omacp, latent, noise)
    return out
